# Initial kernel scaffold; baseline (speedup 1.0000x reference)
import jax
import jax.numpy as jnp
from jax.experimental import pallas as pl


def kernel(atom_feat, bond_idx, graph_idx, bond_feat, W_embed, b_embed, Wf, bf, Ws, bs, W_out, b_out):
    raise NotImplementedError("write your pallas kernel here")


# speedup vs baseline: 2.0193x; 2.0193x over previous
import functools

import jax
import jax.numpy as jnp
from jax import lax
from jax.experimental import pallas as pl
from jax.experimental.pallas import tpu as pltpu
from jax.experimental.pallas import tpu_sc as plsc

_N = 10000
_E = 320000
_H = 128
_NL = 4
_G = 64

_NC = 2
_NS = 16
_CHUNK = 80
_EPT = _E // (_NC * _NS)
_NCHUNK = _EPT // _CHUNK
_STRIPE = 640
_BLK = 80

_LOG1P_C = (
    0.9999987638835109, -0.49987192527791824, 0.331120583677199,
    -0.2351488241025344, 0.14943483647092481, -0.06658820574006942,
    0.0142028592618747,
)


def _log1p01(t):
    p = jnp.full((16,), _LOG1P_C[-1], dtype=jnp.float32)
    for c in _LOG1P_C[-2::-1]:
        p = p * t + c
    return t * p


def _edge_body(src_hbm, dst_hbm, pdf_hbm, pds_hbm, psf_hbm, pss_hbm, eb_hbm,
               out_hbm, idx_s, idx_d, g1, g2, ubuf, msg_v, agg_sh, sem):
    cid = lax.axis_index("c")
    sid = lax.axis_index("s")

    zeros16 = jnp.zeros((16,), jnp.float32)

    def _zero_row(i, carry):
        for j in range(8):
            msg_v[i, pl.ds(16 * j, 16)] = zeros16
        return carry

    lax.fori_loop(0, _BLK, _zero_row, 0)
    nblk = jnp.where(sid == _NS - 1, (_N - (_NS - 1) * _STRIPE) // _BLK,
                     _STRIPE // _BLK)

    def _zero_blk(b, carry):
        pltpu.sync_copy(msg_v, agg_sh.at[pl.ds(sid * _STRIPE + b * _BLK, _BLK)])
        return carry

    lax.fori_loop(0, nblk, _zero_blk, 0)
    plsc.subcore_barrier()

    tile_base = (cid * _NS + sid) * _EPT

    def _chunk(q, carry):
        base = tile_base + q * _CHUNK
        pltpu.sync_copy(src_hbm.at[pl.ds(base, _CHUNK)], idx_s)
        pltpu.sync_copy(dst_hbm.at[pl.ds(base, _CHUNK)], idx_d)
        cp1 = pltpu.async_copy(pdf_hbm.at[idx_d], g1, sem)
        cp2 = pltpu.async_copy(psf_hbm.at[idx_s], g2, sem)
        cp3 = pltpu.async_copy(eb_hbm.at[pl.ds(base, _CHUNK), pl.ds(0, _H)],
                               ubuf, sem)
        cp1.wait()
        cp2.wait()
        cp3.wait()

        def _edge_f(i, c2):
            for j in range(8):
                lo = 16 * j
                uf = (g1[i, pl.ds(lo, 16)] + g2[i, pl.ds(lo, 16)]
                      + ubuf[i, pl.ds(lo, 16)])
                msg_v[i, pl.ds(lo, 16)] = 1.0 / (1.0 + jnp.exp(-uf))
            return c2

        lax.fori_loop(0, _CHUNK, _edge_f, 0)
        cp1 = pltpu.async_copy(pds_hbm.at[idx_d], g1, sem)
        cp2 = pltpu.async_copy(pss_hbm.at[idx_s], g2, sem)
        cp3 = pltpu.async_copy(eb_hbm.at[pl.ds(base, _CHUNK), pl.ds(_H, _H)],
                               ubuf, sem)
        cp1.wait()
        cp2.wait()
        cp3.wait()

        def _edge_s(i, c2):
            for j in range(8):
                lo = 16 * j
                us = (g1[i, pl.ds(lo, 16)] + g2[i, pl.ds(lo, 16)]
                      + ubuf[i, pl.ds(lo, 16)])
                t = jnp.exp(-jnp.abs(us))
                sp = jnp.maximum(us, 0.0) + _log1p01(t)
                msg_v[i, pl.ds(lo, 16)] = msg_v[i, pl.ds(lo, 16)] * sp
            return c2

        lax.fori_loop(0, _CHUNK, _edge_s, 0)
        pltpu.sync_copy(msg_v, agg_sh.at[idx_d], add=True)
        return carry

    lax.fori_loop(0, _NCHUNK, _chunk, 0)
    plsc.subcore_barrier()

    def _out_blk(b, carry):
        r0 = sid * _STRIPE + b * _BLK
        pltpu.sync_copy(agg_sh.at[pl.ds(r0, _BLK)], msg_v)
        pltpu.sync_copy(msg_v, out_hbm.at[cid, pl.ds(r0, _BLK)])
        return carry

    lax.fori_loop(0, nblk, _out_blk, 0)


_edge_pass = pl.kernel(
    _edge_body,
    out_type=jax.ShapeDtypeStruct((_NC, _N, _H), jnp.float32),
    mesh=plsc.VectorSubcoreMesh(core_axis_name="c", subcore_axis_name="s"),
    scratch_types=[
        pltpu.VMEM((_CHUNK,), jnp.int32),
        pltpu.VMEM((_CHUNK,), jnp.int32),
        pltpu.VMEM((_CHUNK, _H), jnp.float32),
        pltpu.VMEM((_CHUNK, _H), jnp.float32),
        pltpu.VMEM((_CHUNK, _H), jnp.float32),
        pltpu.VMEM((_CHUNK, _H), jnp.float32),
        pltpu.VMEM_SHARED((_N, _H), jnp.float32),
        pltpu.SemaphoreType.DMA,
    ],
)


_BM = 1000
_BME = 4000


_P_OUT_SPECS = [
    pl.BlockSpec((_BM, _H), lambda i: (i, 0)),
    pl.BlockSpec((_BM, _H), lambda i: (i, 0)),
    pl.BlockSpec((_BM, _H), lambda i: (i, 0)),
    pl.BlockSpec((_BM, _H), lambda i: (i, 0)),
    pl.BlockSpec((_BM, _H), lambda i: (i, 0)),
]
_P_OUT_SHAPE = [jax.ShapeDtypeStruct((_N, _H), jnp.float32)] * 5


def _split_proj(h, w512, h_ref, pdf_ref, pds_ref, psf_ref, pss_ref):
    h_ref[...] = h
    p = jnp.dot(h, w512, preferred_element_type=jnp.float32)
    pdf_ref[...] = p[:, :_H]
    pds_ref[...] = p[:, _H:2 * _H]
    psf_ref[...] = p[:, 2 * _H:3 * _H]
    pss_ref[...] = p[:, 3 * _H:]


def _stage0_body(atom_ref, wemb_ref, bemb_ref, w512_ref,
                 h_ref, pdf_ref, pds_ref, psf_ref, pss_ref):
    a = atom_ref[...]
    h = jnp.dot(a, wemb_ref[...], preferred_element_type=jnp.float32) + bemb_ref[...]
    h = jnp.where(h > 0, h, jnp.exp(h) - 1.0)
    _split_proj(h, w512_ref[...], h_ref, pdf_ref, pds_ref, psf_ref, pss_ref)


def _stage0(atom_feat, W_embed, b_embed, W512_0):
    return pl.pallas_call(
        _stage0_body,
        grid=(_N // _BM,),
        in_specs=[
            pl.BlockSpec((_BM, _H), lambda i: (i, 0)),
            pl.BlockSpec((_H, _H), lambda i: (0, 0)),
            pl.BlockSpec((1, _H), lambda i: (0, 0)),
            pl.BlockSpec((_H, 4 * _H), lambda i: (0, 0)),
        ],
        out_specs=_P_OUT_SPECS,
        out_shape=_P_OUT_SHAPE,
    )(atom_feat, W_embed, b_embed, W512_0)


def _stageL_body(h_ref, agg_ref, w512_ref,
                 h_out_ref, pdf_ref, pds_ref, psf_ref, pss_ref):
    h = h_ref[...] + agg_ref[0] + agg_ref[1]
    _split_proj(h, w512_ref[...], h_out_ref, pdf_ref, pds_ref, psf_ref, pss_ref)


def _stageL(h, aggp, W512_l):
    return pl.pallas_call(
        _stageL_body,
        grid=(_N // _BM,),
        in_specs=[
            pl.BlockSpec((_BM, _H), lambda i: (i, 0)),
            pl.BlockSpec((_NC, _BM, _H), lambda i: (0, i, 0)),
            pl.BlockSpec((_H, 4 * _H), lambda i: (0, 0)),
        ],
        out_specs=_P_OUT_SPECS,
        out_shape=_P_OUT_SHAPE,
    )(h, aggp, W512_l)


def _bond_body(bfeat_ref, w_ref, b_ref, out_ref):
    out_ref[...] = (jnp.dot(bfeat_ref[...], w_ref[...],
                            preferred_element_type=jnp.float32) + b_ref[...])


def _bond(bond_feat, Wbond_l, bias_l):
    return pl.pallas_call(
        _bond_body,
        grid=(_E // _BME,),
        in_specs=[
            pl.BlockSpec((_BME, 16), lambda i: (i, 0)),
            pl.BlockSpec((16, 2 * _H), lambda i: (0, 0)),
            pl.BlockSpec((1, 2 * _H), lambda i: (0, 0)),
        ],
        out_specs=pl.BlockSpec((_BME, 2 * _H), lambda i: (i, 0)),
        out_shape=jax.ShapeDtypeStruct((_E, 2 * _H), jnp.float32),
    )(bond_feat, Wbond_l, bias_l)


def _pool_body(h_ref, agg_ref, gidx_ref, wout_ref, bout_ref, out_ref):
    h = h_ref[...] + agg_ref[0] + agg_ref[1]
    g = gidx_ref[...]
    iota = lax.broadcasted_iota(jnp.int32, (_G, _N), 0)
    onehot = (iota == g).astype(jnp.float32)
    sums = jnp.dot(onehot, h, preferred_element_type=jnp.float32)
    counts = jnp.sum(onehot, axis=1, keepdims=True)
    pooled = sums / jnp.maximum(counts, 1.0)
    out_ref[...] = (jnp.dot(pooled, wout_ref[...],
                            preferred_element_type=jnp.float32) + bout_ref[...])


def _pool(h, aggp, gidx, W_out, b_out):
    return pl.pallas_call(
        _pool_body,
        out_shape=jax.ShapeDtypeStruct((_G, _H), jnp.float32),
    )(h, aggp, gidx, W_out, b_out)


def kernel(atom_feat, bond_idx, graph_idx, bond_feat, W_embed, b_embed,
           Wf, bf, Ws, bs, W_out, b_out):
    src = bond_idx[0].astype(jnp.int32)
    dst = bond_idx[1].astype(jnp.int32)
    gidx = graph_idx.astype(jnp.int32).reshape(1, _N)

    W512 = jnp.concatenate(
        [Wf[:, :_H, :], Ws[:, :_H, :], Wf[:, _H:2 * _H, :], Ws[:, _H:2 * _H, :]],
        axis=2)
    Wbond = jnp.concatenate([Wf[:, 2 * _H:, :], Ws[:, 2 * _H:, :]], axis=2)
    bias = jnp.concatenate([bf, bs], axis=1).reshape(_NL, 1, 2 * _H)

    h, pdf, pds, psf, pss = _stage0(atom_feat, W_embed,
                                    b_embed.reshape(1, _H), W512[0])
    aggp = None
    for l in range(_NL):
        eb = _bond(bond_feat, Wbond[l], bias[l])
        aggp = _edge_pass(src, dst, pdf, pds, psf, pss, eb)
        if l + 1 < _NL:
            h, pdf, pds, psf, pss = _stageL(h, aggp, W512[l + 1])
    return _pool(h, aggp, gidx, W_out, b_out.reshape(1, _H))
